# baseline (speedup 1.0000x reference)
import functools

import jax
import jax.numpy as jnp
from jax import lax
from jax.experimental import pallas as pl
from jax.experimental.pallas import tpu as pltpu
from jax.experimental.pallas import tpu_sc as plsc

B, C, H, W = 8, 768, 56, 56
ROWS = B * H * W
TR = ROWS // 8
D = 6 * 8 * 128
NC, NS = 2, 16
NW = NC * NS
TPW = TR // NW
TCH = 2
NCH = TPW // TCH
KG = C // 16


def _sc_body(z_hbm, idx_hbm, out_hbm, idx_v, in0, in1, ou0, ou1,
             si0, si1, so0, so1):
    wid = lax.axis_index("c") * NS + lax.axis_index("s")
    base = wid * TPW
    pltpu.sync_copy(idx_hbm, idx_v)
    fidx = []
    for k in range(KG):
        v = idx_v[pl.ds(16 * k, 16)]
        fidx.append(((v >> 7) << 10) + (v & 127))
    ins = (in0, in1)
    outs = (ou0, ou1)
    sis = (si0, si1)
    sos = (so0, so1)

    def start_in(c, par):
        pltpu.async_copy(z_hbm.at[pl.ds(base + c * TCH, TCH)],
                         ins[par], sis[par])

    def wait_in(par):
        pltpu.make_async_copy(z_hbm.at[pl.ds(base, TCH)],
                              ins[par], sis[par]).wait()

    def start_out(c, par):
        pltpu.async_copy(outs[par], out_hbm.at[pl.ds(base + c * TCH, TCH)],
                         sos[par])

    def wait_out(par):
        pltpu.make_async_copy(outs[par],
                              out_hbm.at[pl.ds(base, TCH)], sos[par]).wait()

    def compute_chunk(cc, par):
        ib, ob = ins[par], outs[par]
        wait_in(par)

        @pl.when(cc >= 2)
        def _():
            wait_out(par)

        @pl.loop(0, TCH * 8)
        def _row(i):
            rt = i >> 3
            roff = (i & 7) << 7
            rtv = jnp.full((16,), rt, dtype=jnp.int32)
            for k in range(KG):
                val = plsc.load_gather(ib, [rtv, fidx[k] + roff])
                ob[rt, pl.ds(roff + (k // 8) * 1024 + (k % 8) * 16, 16)] = val

        start_out(cc, par)

        @pl.when(cc + 2 < NCH)
        def _():
            start_in(cc + 2, par)

    start_in(0, 0)
    start_in(1, 1)

    @pl.loop(0, NCH - 1, step=2)
    def _chunk(c):
        for par in range(2):
            compute_chunk(c + par, par)

    compute_chunk(NCH - 1, 0)
    wait_out(0)
    wait_out(1)


def kernel(x, forward_shuffle_idx):
    x_t = jnp.transpose(x, (0, 2, 3, 1)).reshape(ROWS, C)
    z = jnp.transpose(x_t.reshape(TR, 8, C // 128, 128),
                      (0, 2, 1, 3)).reshape(TR, D)
    idx32 = forward_shuffle_idx.astype(jnp.int32)
    run = functools.partial(
        pl.kernel,
        mesh=plsc.VectorSubcoreMesh(core_axis_name="c", subcore_axis_name="s"),
        out_type=jax.ShapeDtypeStruct((TR, D), jnp.float32),
        scratch_types=[
            pltpu.VMEM((C,), jnp.int32),
            pltpu.VMEM((TCH, D), jnp.float32),
            pltpu.VMEM((TCH, D), jnp.float32),
            pltpu.VMEM((TCH, D), jnp.float32),
            pltpu.VMEM((TCH, D), jnp.float32),
            pltpu.SemaphoreType.DMA,
            pltpu.SemaphoreType.DMA,
            pltpu.SemaphoreType.DMA,
            pltpu.SemaphoreType.DMA,
        ],
        compiler_params=pltpu.CompilerParams(use_tc_tiling_on_sc=False,
                                             needs_layout_passes=False),
    )(_sc_body)
    out_z = run(z, idx32)
    out_t = jnp.transpose(out_z.reshape(TR, C // 128, 8, 128),
                          (0, 2, 1, 3)).reshape(ROWS, C)
    return jnp.transpose(out_t.reshape(B, H, W, C), (0, 3, 1, 2))
